# baseline (speedup 1.0000x reference)
import functools

import jax
import jax.numpy as jnp
from jax import lax
from jax.experimental import pallas as pl
from jax.experimental.pallas import tpu as pltpu
from jax.experimental.pallas import tpu_sc as plsc

N_SAMPLES = 3145728
NUM_RAYS_C = 65536
NC = 2
NS = 16
NW = NC * NS
RPT = NUM_RAYS_C // NW
OUTW = RPT * 3
L = 16
B = 128
CHUNK = 512
GB = N_SAMPLES // B
SEARCH_ITERS = 15


def _sc_body(colors_hbm, weights_hbm, rays_hbm, lo16_hbm, hi16_hbm, out_hbm,
             ray_v, w_v, c_v, acc_v, probe_v, clip_v,
             sem0, sem1, sem2, sem3, sem4, sem5):
    cid = lax.axis_index("c")
    sid = lax.axis_index("s")
    wid = cid * NS + sid
    base = wid * RPT

    def probe(mid, dst):
        pltpu.sync_copy(rays_hbm.at[pl.ds(pl.multiple_of(mid * L, L), L)],
                        dst)

    def it(_, carry):
        lo0, hi0, lo1, hi1 = carry
        mid0 = jnp.minimum((lo0 + hi0) // 2, GB - 1)
        mid1 = jnp.minimum((lo1 + hi1) // 2, GB - 1)
        cp0 = pltpu.make_async_copy(
            rays_hbm.at[pl.ds(pl.multiple_of(mid0 * B, L), L)],
            probe_v.at[pl.ds(0, L)], sem0)
        cp1 = pltpu.make_async_copy(
            rays_hbm.at[pl.ds(pl.multiple_of(mid1 * B, L), L)],
            probe_v.at[pl.ds(L, L)], sem1)
        cp0.start()
        cp1.start()
        cp0.wait()
        cp1.wait()
        go0 = probe_v[pl.ds(0, L)][0] >= base
        go1 = probe_v[pl.ds(L, L)][0] >= base + RPT
        return (jnp.where(go0, lo0, mid0 + 1), jnp.where(go0, mid0, hi0),
                jnp.where(go1, lo1, mid1 + 1), jnp.where(go1, mid1, hi1))

    g0, _, g1, _ = lax.fori_loop(
        0, SEARCH_ITERS, it,
        (jnp.int32(0), jnp.int32(GB), jnp.int32(0), jnp.int32(GB)))
    lo_a = jnp.maximum(g0 - 1, 0) * B
    hi_s = g1 * B

    zeros16 = jnp.zeros((L,), jnp.float32)

    def zero_it(j, _):
        acc_v[pl.ds(j * L, L)] = zeros16
        return 0
    lax.fori_loop(0, OUTW // L, zero_it, 0)

    iota = lax.iota(jnp.int32, L)

    nsamp = hi_s - lo_a
    nchunks = (nsamp + CHUNK - 1) // CHUNK

    sems = ((sem0, sem1, sem2), (sem3, sem4, sem5))

    def chunk_start(ci):
        start_nom = lo_a + ci * CHUNK
        start = pl.multiple_of(jnp.minimum(start_nom, N_SAMPLES - CHUNK), B)
        return start_nom, start

    def copies(ci, par):
        _, start = chunk_start(ci)
        boff = par * CHUNK
        sr, sw, sc = sems[par]
        return (
            pltpu.make_async_copy(rays_hbm.at[pl.ds(start, CHUNK)],
                                  ray_v.at[pl.ds(boff, CHUNK)], sr),
            pltpu.make_async_copy(weights_hbm.at[pl.ds(start, CHUNK)],
                                  w_v.at[pl.ds(boff, CHUNK)], sw),
            pltpu.make_async_copy(
                colors_hbm.at[pl.ds(pl.multiple_of(start * 3, 8), 3 * CHUNK)],
                c_v.at[pl.ds(3 * boff, 3 * CHUNK)], sc),
        )

    def issue(ci, par):
        for cp in copies(ci, par):
            cp.start()

    SUB = CHUNK // L

    def compute(ci, par):
        boff = par * CHUNK
        start_nom, start = chunk_start(ci)
        a_s = start_nom - start + boff
        lane_base = iota * SUB + boff

        def step(i, _):
            u = (iota + i) & (SUB - 1)
            sv = lane_base + u
            rv = plsc.load_gather(ray_v, [sv])
            wv = plsc.load_gather(w_v, [sv])
            dv = rv - base
            du = plsc.bitcast(dv, jnp.uint32)
            mask = (du < jnp.uint32(RPT)) & (sv >= a_s)
            t0 = (dv >> 7) * (3 * B) + (dv & (B - 1))
            cb = (sv >> 7) * (3 * B) + (sv & (B - 1))
            for ch in range(3):
                cv = plsc.load_gather(c_v, [cb if ch == 0 else cb + ch * B])
                tk = t0 if ch == 0 else t0 + ch * B
                plsc.addupdate_scatter(acc_v, [tk], cv * wv, mask=mask)
            return 0
        lax.fori_loop(0, SUB, step, 0, unroll=16)

    @pl.when(nchunks > 0)
    def _prime():
        issue(0, 0)

    def pair_body(c, _):
        for b in range(2):
            ci = 2 * c + b

            @pl.when(ci < nchunks)
            def _step():
                for cp in copies(ci, b):
                    cp.wait()

                @pl.when(ci + 1 < nchunks)
                def _prefetch():
                    issue(ci + 1, 1 - b)

                compute(ci, b)
        return 0

    lax.fori_loop(0, (nchunks + 1) // 2, pair_body, 0)

    pltpu.sync_copy(lo16_hbm, clip_v)
    lo_vec = clip_v[...]
    pltpu.sync_copy(hi16_hbm, clip_v)
    hi_vec = clip_v[...]

    def clip_it(j, _):
        v = acc_v[pl.ds(j * L, L)]
        acc_v[pl.ds(j * L, L)] = jnp.minimum(jnp.maximum(v, lo_vec), hi_vec)
        return 0
    lax.fori_loop(0, OUTW // L, clip_it, 0)

    pltpu.sync_copy(acc_v, out_hbm.at[pl.ds(wid * OUTW, OUTW)])


@functools.partial(
    pl.kernel,
    out_type=jax.ShapeDtypeStruct((NUM_RAYS_C * 3,), jnp.float32),
    mesh=plsc.VectorSubcoreMesh(core_axis_name="c", subcore_axis_name="s",
                                num_cores=NC, num_subcores=NS),
    compiler_params=pltpu.CompilerParams(needs_layout_passes=False),
    scratch_types=[
        pltpu.VMEM((2 * CHUNK,), jnp.int32),
        pltpu.VMEM((2 * CHUNK,), jnp.float32),
        pltpu.VMEM((6 * CHUNK,), jnp.float32),
        pltpu.VMEM((OUTW,), jnp.float32),
        pltpu.VMEM((2 * L,), jnp.int32),
        pltpu.VMEM((L,), jnp.float32),
        pltpu.SemaphoreType.DMA,
        pltpu.SemaphoreType.DMA,
        pltpu.SemaphoreType.DMA,
        pltpu.SemaphoreType.DMA,
        pltpu.SemaphoreType.DMA,
        pltpu.SemaphoreType.DMA,
    ],
)
def _sc_segment_rgb(colors_hbm, weights_hbm, rays_hbm, lo16_hbm, hi16_hbm,
                    out_hbm, ray_v, w_v, c_v, acc_v, probe_v, clip_v,
                    sem0, sem1, sem2, sem3, sem4, sem5):
    _sc_body(colors_hbm, weights_hbm, rays_hbm, lo16_hbm, hi16_hbm, out_hbm,
             ray_v, w_v, c_v, acc_v, probe_v, clip_v,
             sem0, sem1, sem2, sem3, sem4, sem5)


def kernel(colors, weights, min, max, ray_indices, num_rays):
    cpl = (colors.reshape(N_SAMPLES // B, B, 3)
           .transpose(0, 2, 1)
           .reshape(3 * N_SAMPLES))
    wf = weights.reshape(-1)
    ri = ray_indices.astype(jnp.int32)
    lo16 = jnp.full((L,), min, dtype=jnp.float32)
    hi16 = jnp.full((L,), max, dtype=jnp.float32)
    out = _sc_segment_rgb(cpl, wf, ri, lo16, hi16)
    return (out.reshape(NUM_RAYS_C // B, 3, B)
            .transpose(0, 2, 1)
            .reshape(NUM_RAYS_C, 3))


# speedup vs baseline: 1.1131x; 1.1131x over previous
import functools

import jax
import jax.numpy as jnp
from jax import lax
from jax.experimental import pallas as pl
from jax.experimental.pallas import tpu as pltpu
from jax.experimental.pallas import tpu_sc as plsc

N_SAMPLES = 3145728
NUM_RAYS_C = 65536
NC = 2
NS = 16
NW = NC * NS
RPT = NUM_RAYS_C // NW
OUTW = RPT * 3
L = 16
B = 128
CHUNK = 1024
GB = N_SAMPLES // B
SEARCH_ITERS = 15


def _sc_body(colors_hbm, weights_hbm, rays_hbm, lo16_hbm, hi16_hbm, out_hbm,
             ray_v, w_v, c_v, acc_v, probe_v, clip_v,
             sem0, sem1, sem2, sem3, sem4, sem5):
    cid = lax.axis_index("c")
    sid = lax.axis_index("s")
    wid = cid * NS + sid
    base = wid * RPT

    def it(_, carry):
        lo0, hi0, lo1, hi1 = carry
        mid0 = jnp.minimum((lo0 + hi0) // 2, GB - 1)
        mid1 = jnp.minimum((lo1 + hi1) // 2, GB - 1)
        cp0 = pltpu.make_async_copy(
            rays_hbm.at[pl.ds(pl.multiple_of(mid0 * B, L), L)],
            probe_v.at[pl.ds(0, L)], sem0)
        cp1 = pltpu.make_async_copy(
            rays_hbm.at[pl.ds(pl.multiple_of(mid1 * B, L), L)],
            probe_v.at[pl.ds(L, L)], sem1)
        cp0.start()
        cp1.start()
        cp0.wait()
        cp1.wait()
        go0 = probe_v[pl.ds(0, L)][0] >= base
        go1 = probe_v[pl.ds(L, L)][0] >= base + RPT
        return (jnp.where(go0, lo0, mid0 + 1), jnp.where(go0, mid0, hi0),
                jnp.where(go1, lo1, mid1 + 1), jnp.where(go1, mid1, hi1))

    g0, _, g1, _ = lax.fori_loop(
        0, SEARCH_ITERS, it,
        (jnp.int32(0), jnp.int32(GB), jnp.int32(0), jnp.int32(GB)))
    lo_a = jnp.maximum(g0 - 1, 0) * B
    hi_s = g1 * B

    zeros16 = jnp.zeros((L,), jnp.float32)

    def zero_it(j, _):
        acc_v[pl.ds(j * L, L)] = zeros16
        return 0
    lax.fori_loop(0, OUTW // L, zero_it, 0)

    iota = lax.iota(jnp.int32, L)

    nsamp = hi_s - lo_a
    nchunks = (nsamp + CHUNK - 1) // CHUNK

    sems = ((sem0, sem1, sem2), (sem3, sem4, sem5))

    def chunk_start(ci):
        start_nom = lo_a + ci * CHUNK
        start = pl.multiple_of(jnp.minimum(start_nom, N_SAMPLES - CHUNK), B)
        return start_nom, start

    def copies(ci, par):
        _, start = chunk_start(ci)
        boff = par * CHUNK
        sr, sw, sc = sems[par]
        return (
            pltpu.make_async_copy(rays_hbm.at[pl.ds(start, CHUNK)],
                                  ray_v.at[pl.ds(boff, CHUNK)], sr),
            pltpu.make_async_copy(weights_hbm.at[pl.ds(start, CHUNK)],
                                  w_v.at[pl.ds(boff, CHUNK)], sw),
            pltpu.make_async_copy(
                colors_hbm.at[pl.ds(pl.multiple_of(start * 3, 8), 3 * CHUNK)],
                c_v.at[pl.ds(3 * boff, 3 * CHUNK)], sc),
        )

    def issue(ci, par):
        for cp in copies(ci, par):
            cp.start()

    SUB = CHUNK // L

    def compute(ci, par):
        boff = par * CHUNK
        start_nom, start = chunk_start(ci)
        a_s = start_nom - start + boff
        lane_base = iota * SUB + boff

        def step(i, _):
            u = (iota + i) & (SUB - 1)
            sv = lane_base + u
            rv = plsc.load_gather(ray_v, [sv])
            wv = plsc.load_gather(w_v, [sv])
            dv = rv - base
            du = plsc.bitcast(dv, jnp.uint32)
            mask = (du < jnp.uint32(RPT)) & (sv >= a_s)
            t0 = (dv >> 7) * (3 * B) + (dv & (B - 1))
            cb = (sv >> 7) * (3 * B) + (sv & (B - 1))
            for ch in range(3):
                cv = plsc.load_gather(c_v, [cb if ch == 0 else cb + ch * B])
                tk = t0 if ch == 0 else t0 + ch * B
                plsc.addupdate_scatter(acc_v, [tk], cv * wv, mask=mask)
            return 0
        lax.fori_loop(0, SUB, step, 0, unroll=16)

    @pl.when(nchunks > 0)
    def _prime():
        issue(0, 0)

    def pair_body(c, _):
        for b in range(2):
            ci = 2 * c + b

            @pl.when(ci < nchunks)
            def _step():
                for cp in copies(ci, b):
                    cp.wait()

                @pl.when(ci + 1 < nchunks)
                def _prefetch():
                    issue(ci + 1, 1 - b)

                compute(ci, b)
        return 0

    lax.fori_loop(0, (nchunks + 1) // 2, pair_body, 0)

    pltpu.sync_copy(lo16_hbm, clip_v)
    lo_vec = clip_v[...]
    pltpu.sync_copy(hi16_hbm, clip_v)
    hi_vec = clip_v[...]

    def clip_it(j, _):
        v = acc_v[pl.ds(j * L, L)]
        acc_v[pl.ds(j * L, L)] = jnp.minimum(jnp.maximum(v, lo_vec), hi_vec)
        return 0
    lax.fori_loop(0, OUTW // L, clip_it, 0)

    pltpu.sync_copy(acc_v, out_hbm.at[pl.ds(wid * OUTW, OUTW)])


@functools.partial(
    pl.kernel,
    out_type=jax.ShapeDtypeStruct((NUM_RAYS_C * 3,), jnp.float32),
    mesh=plsc.VectorSubcoreMesh(core_axis_name="c", subcore_axis_name="s",
                                num_cores=NC, num_subcores=NS),
    compiler_params=pltpu.CompilerParams(needs_layout_passes=False),
    scratch_types=[
        pltpu.VMEM((2 * CHUNK,), jnp.int32),
        pltpu.VMEM((2 * CHUNK,), jnp.float32),
        pltpu.VMEM((6 * CHUNK,), jnp.float32),
        pltpu.VMEM((OUTW,), jnp.float32),
        pltpu.VMEM((2 * L,), jnp.int32),
        pltpu.VMEM((L,), jnp.float32),
        pltpu.SemaphoreType.DMA,
        pltpu.SemaphoreType.DMA,
        pltpu.SemaphoreType.DMA,
        pltpu.SemaphoreType.DMA,
        pltpu.SemaphoreType.DMA,
        pltpu.SemaphoreType.DMA,
    ],
)
def _sc_segment_rgb(colors_hbm, weights_hbm, rays_hbm, lo16_hbm, hi16_hbm,
                    out_hbm, ray_v, w_v, c_v, acc_v, probe_v, clip_v,
                    sem0, sem1, sem2, sem3, sem4, sem5):
    _sc_body(colors_hbm, weights_hbm, rays_hbm, lo16_hbm, hi16_hbm, out_hbm,
             ray_v, w_v, c_v, acc_v, probe_v, clip_v,
             sem0, sem1, sem2, sem3, sem4, sem5)


def kernel(colors, weights, min, max, ray_indices, num_rays):
    cpl = (colors.reshape(N_SAMPLES // B, B, 3)
           .transpose(0, 2, 1)
           .reshape(3 * N_SAMPLES))
    wf = weights.reshape(-1)
    ri = ray_indices.astype(jnp.int32)
    lo16 = jnp.full((L,), min, dtype=jnp.float32)
    hi16 = jnp.full((L,), max, dtype=jnp.float32)
    out = _sc_segment_rgb(cpl, wf, ri, lo16, hi16)
    return (out.reshape(NUM_RAYS_C // B, 3, B)
            .transpose(0, 2, 1)
            .reshape(NUM_RAYS_C, 3))
